# baseline (speedup 1.0000x reference)
import jax
import jax.numpy as jnp
from jax import lax
from jax.experimental import pallas as pl
from jax.experimental.pallas import tpu as pltpu
from jax.experimental.pallas import tpu_sc as plsc

VSZ = 1000000
DSZ = 64
NI = 16384
NJ = 20
B_TOTAL = NI * NJ

NC = 2
NS = 16
NW = NC * NS
IB = NI // NW
B_PER_W = IB * NJ
CHUNK = 128
NCHUNK = B_PER_W // CHUNK
NBUF = 4
G = 3


def _emb_body(xt_hbm, w_hbm, out_hbm, idx_v, gidx, rows, gsem, ssem):
    wid = lax.axis_index("s") * NC + lax.axis_index("c")
    base = wid * B_PER_W

    pltpu.sync_copy(xt_hbm.at[:, pl.ds(wid * IB, IB)], idx_v)

    iota = lax.iota(jnp.int32, 16)
    zeros_i = jnp.zeros((16,), jnp.int32)
    zeros16 = jnp.zeros((16,), jnp.float32)

    def build(k, b):
        for g in range(8):
            n = iota + (k * CHUNK + g * 16)
            p = n // NJ
            j = n - p * NJ
            cvec = plsc.load_gather(idx_v, [j, p])
            gidx[b, pl.ds(g * 16, 16)] = cvec

    def fire(b):
        pltpu.async_copy(w_hbm.at[gidx.at[b]], rows.at[b], gsem.at[b])

    def wait_gather(b):
        pltpu.make_async_copy(w_hbm.at[gidx.at[b]], rows.at[b],
                              gsem.at[b]).wait()

    def store(k, b):
        pltpu.async_copy(rows.at[b],
                         out_hbm.at[pl.ds(base + k * CHUNK, CHUNK)],
                         ssem.at[b])

    def wait_store(b):
        pltpu.make_async_copy(rows.at[b], out_hbm.at[pl.ds(base, CHUNK)],
                              ssem.at[b]).wait()

    def fixup(b):
        m_any = gidx[b, pl.ds(0, 16)] == 0
        for v in range(1, CHUNK // 16):
            m_any = m_any | (gidx[b, pl.ds(v * 16, 16)] == 0)
        mi = jnp.where(m_any, zeros_i + 1, zeros_i)
        npad = mi[0]
        for l in range(1, 16):
            npad = npad | mi[l]

        @pl.when(npad > 0)
        def _fix():
            def per_vreg(v, carry):
                iv = gidx[b, pl.ds(v * 16, 16)]
                for l in range(16):
                    @pl.when(iv[l] == 0)
                    def _zero_row(v=v, l=l):
                        for cc in range(DSZ // 16):
                            rows[b, v * 16 + l, pl.ds(cc * 16, 16)] = zeros16
                return carry

            lax.fori_loop(0, CHUNK // 16, per_vreg, 0)

    for k in range(G):
        build(k, k % NBUF)
        fire(k % NBUF)

    def stage_body(s, carry):
        kb = s * NBUF
        for b in range(NBUF):
            k = kb + b
            wait_gather(b)
            fixup(b)
            store(k, b)
            bg = (b + G) % NBUF
            kg = k + G

            @pl.when(kg < NCHUNK)
            def _prefetch(kg=kg, bg=bg):
                @pl.when(kg >= NBUF)
                def _drain(bg=bg):
                    wait_store(bg)
                build(kg, bg)
                fire(bg)
        return carry

    lax.fori_loop(0, NCHUNK // NBUF, stage_body, 0)

    for i in range(G):
        wait_store((NCHUNK - G + i) % NBUF)


@jax.jit
def _emb_lookup(xt, w):
    mesh = plsc.VectorSubcoreMesh(core_axis_name="c", subcore_axis_name="s")
    return pl.kernel(
        _emb_body,
        out_type=jax.ShapeDtypeStruct((B_TOTAL, DSZ), jnp.float32),
        mesh=mesh,
        compiler_params=pltpu.CompilerParams(use_tc_tiling_on_sc=False,
                                             needs_layout_passes=False),
        scratch_types=[
            pltpu.VMEM((NJ, IB), jnp.int32),
            pltpu.VMEM((NBUF, CHUNK), jnp.int32),
            pltpu.VMEM((NBUF, CHUNK, DSZ), jnp.float32),
            pltpu.SemaphoreType.DMA((NBUF,)),
            pltpu.SemaphoreType.DMA((NBUF,)),
        ],
    )(xt, w)


def kernel(x, W):
    xt = x.T.astype(jnp.int32)
    out = _emb_lookup(xt, W)
    return out.reshape(NI, NJ, DSZ)
